# baseline (speedup 1.0000x reference)
import functools

import numpy as _np

import jax
import jax.numpy as jnp
from jax import lax
from jax.experimental import pallas as pl
from jax.experimental.pallas import tpu as pltpu
from jax.experimental.pallas import tpu_sc as plsc

N = 10000
E = 160000
DN = 128
DE = 16
H = 32
L = 3
B = 64
STEPS = 6

NP_ = 10240
NC = 2
NS = 16
NW = NC * NS
E_PAD = 163840
EPW = E_PAD // NW
CHUNK = 128
NCHUNK = EPW // CHUNK
NBUF = 5
NBUF_S = 2
TE = 2048
TN = 1024
ROWS_PER_TILE = NP_ // NS

@functools.lru_cache(maxsize=None)
def _sc_kernels():
    mesh = plsc.VectorSubcoreMesh(core_axis_name="c", subcore_axis_name="s")

    @functools.partial(
        pl.kernel, mesh=mesh,
        out_type=jax.ShapeDtypeStruct((E_PAD, DN), jnp.float32),
        scratch_types=[
            pltpu.VMEM((NCHUNK, CHUNK), jnp.int32),
            pltpu.VMEM((NBUF, CHUNK, DN), jnp.float32),
            pltpu.SemaphoreType.DMA,
            pltpu.SemaphoreType.DMA,
        ],
    )
    def sc_gather(h_hbm, src_hbm, out_hbm, idx_all, bufs, sem_g, sem_o):
        wid = lax.axis_index("s") * NC + lax.axis_index("c")
        base = wid * EPW
        pltpu.sync_copy(src_hbm.at[wid], idx_all)

        def body(g, carry):
            j0 = g * NBUF
            cps = [
                pltpu.async_copy(
                    h_hbm.at[idx_all.at[j0 + b]],
                    bufs.at[b], sem_g)
                for b in range(NBUF)
            ]
            outs = []
            for b in range(NBUF):
                cps[b].wait()
                outs.append(pltpu.async_copy(
                    bufs.at[b],
                    out_hbm.at[pl.ds(base + (j0 + b) * CHUNK, CHUNK)],
                    sem_o))
            for o in outs:
                o.wait()
            return carry

        lax.fori_loop(0, NCHUNK // NBUF, body, 0)

    @functools.partial(
        pl.kernel, mesh=mesh,
        out_type=jax.ShapeDtypeStruct((NC * NP_, DN), jnp.float32),
        scratch_types=[
            pltpu.VMEM((NCHUNK, CHUNK), jnp.int32),
            pltpu.VMEM((NBUF_S, CHUNK, DN), jnp.float32),
            pltpu.VMEM_SHARED((NP_, DN), jnp.float32),
            pltpu.SemaphoreType.DMA,
            pltpu.SemaphoreType.DMA,
        ],
    )
    def sc_scatter(msg_hbm, dst_hbm, zeros_hbm, out_hbm, idx_all, bufs,
                   acc_sh, sem_l, sem_s):
        cid = lax.axis_index("c")
        sid = lax.axis_index("s")
        wid = sid * NC + cid
        r0 = sid * ROWS_PER_TILE
        pltpu.sync_copy(zeros_hbm.at[pl.ds(r0, ROWS_PER_TILE)],
                        acc_sh.at[pl.ds(r0, ROWS_PER_TILE)])
        base = wid * EPW
        pltpu.sync_copy(dst_hbm.at[wid], idx_all)
        plsc.subcore_barrier()

        def body(g, carry):
            j0 = g * NBUF_S
            cps = [
                pltpu.async_copy(
                    msg_hbm.at[pl.ds(base + (j0 + b) * CHUNK, CHUNK)],
                    bufs.at[b], sem_l)
                for b in range(NBUF_S)
            ]
            adds = []
            for b in range(NBUF_S):
                cps[b].wait()
                adds.append(pltpu.async_copy(
                    bufs.at[b],
                    acc_sh.at[idx_all.at[j0 + b]],
                    sem_s, add=True))
            for a in adds:
                a.wait()
            return carry

        lax.fori_loop(0, NCHUNK // NBUF_S, body, 0)
        plsc.subcore_barrier()
        pltpu.sync_copy(acc_sh.at[pl.ds(r0, ROWS_PER_TILE)],
                        out_hbm.at[pl.ds(cid * NP_ + r0, ROWS_PER_TILE)])

    return sc_gather, sc_scatter


def _pad_wide(v):
    return jnp.concatenate(
        [v, jnp.zeros((v.shape[0], DN - H), jnp.float32)], axis=1)


def _h0_body(x_ref, w_ref, b_ref, o_ref):
    o_ref[...] = _pad_wide(jnp.dot(x_ref[...], w_ref[...],
                                   preferred_element_type=jnp.float32)
                           + b_ref[...])


def _msg_body(hs_ref, ea_ref, w_ref, r_ref, s_ref, o_ref):
    bf = jnp.bfloat16
    hs2 = jnp.dot(hs_ref[:, :H].astype(bf), w_ref[...].astype(bf),
                  preferred_element_type=jnp.float32)
    eax = jnp.dot(ea_ref[...].astype(bf), r_ref[...].astype(bf),
                  preferred_element_type=jnp.float32)
    msg = jnp.dot((eax * hs2).astype(bf), s_ref[...].astype(bf),
                  preferred_element_type=jnp.float32)
    o_ref[...] = _pad_wide(msg)


def _node_body(agg0_ref, agg1_ref, h_ref, rw_ref, cb_ref, wih_ref, whh_ref,
               bih_ref, bhh_ref, o_ref):
    h = h_ref[:, :H]
    agg = agg0_ref[:, :H] + agg1_ref[:, :H]
    m = jnp.maximum(agg + jnp.dot(h, rw_ref[...],
                                  preferred_element_type=jnp.float32)
                    + cb_ref[...], 0.0)
    gi = jnp.dot(m, wih_ref[...], preferred_element_type=jnp.float32) + bih_ref[...]
    gh = jnp.dot(h, whh_ref[...], preferred_element_type=jnp.float32) + bhh_ref[...]
    r = jax.nn.sigmoid(gi[:, :H] + gh[:, :H])
    z = jax.nn.sigmoid(gi[:, H:2 * H] + gh[:, H:2 * H])
    n = jnp.tanh(gi[:, 2 * H:] + r * gh[:, 2 * H:])
    o_ref[...] = _pad_wide((1.0 - z) * n + z * h)


_NEG = -3.0e38


def _s2s_body(h_ref, bt_ref, wih_ref, whh_ref, bih_ref, bhh_ref,
              wo_ref, bo_ref, o_ref):
    h = h_ref[:, :H]
    bt = bt_ref[...]
    mm = (bt == lax.broadcasted_iota(jnp.int32, (NP_, B), 1)).astype(jnp.float32)
    q_star = jnp.zeros((B, 2 * H), jnp.float32)
    hl = jnp.zeros((B, H), jnp.float32)
    cl = jnp.zeros((B, H), jnp.float32)
    for _ in range(STEPS):
        gates = (jnp.dot(q_star, wih_ref[...], preferred_element_type=jnp.float32)
                 + bih_ref[...]
                 + jnp.dot(hl, whh_ref[...], preferred_element_type=jnp.float32)
                 + bhh_ref[...])
        gi = jax.nn.sigmoid(gates[:, :H])
        gf = jax.nn.sigmoid(gates[:, H:2 * H])
        gg = jnp.tanh(gates[:, 2 * H:3 * H])
        go = jax.nn.sigmoid(gates[:, 3 * H:])
        cl = gf * cl + gi * gg
        hl = go * jnp.tanh(cl)
        q = hl
        hq = lax.dot_general(h, q, (((1,), (1,)), ((), ())),
                             preferred_element_type=jnp.float32)
        e = jnp.sum(mm * hq, axis=1, keepdims=True)
        emax = jnp.max(jnp.where(mm > 0, e, _NEG), axis=0, keepdims=True)
        emax = jnp.where(emax > _NEG * 0.5, emax, 0.0)
        ex = jnp.exp(e - jnp.sum(mm * emax, axis=1, keepdims=True))
        denom = jnp.sum(mm * ex, axis=0, keepdims=True)
        a = ex / (jnp.sum(mm * denom, axis=1, keepdims=True) + 1e-16)
        r_ = lax.dot_general(mm * a, h, (((0,), (0,)), ((), ())),
                             preferred_element_type=jnp.float32)
        q_star = jnp.concatenate([q, r_], axis=-1)
    o_ref[...] = (jnp.dot(q_star, wo_ref[...],
                          preferred_element_type=jnp.float32) + bo_ref[...])


def _h0_call(x_p, W0, b0):
    return pl.pallas_call(
        _h0_body,
        grid=(NP_ // TN,),
        in_specs=[pl.BlockSpec((TN, DN), lambda i: (i, 0)),
                  pl.BlockSpec((DN, H), lambda i: (0, 0)),
                  pl.BlockSpec((1, H), lambda i: (0, 0))],
        out_specs=pl.BlockSpec((TN, DN), lambda i: (i, 0)),
        out_shape=jax.ShapeDtypeStruct((NP_, DN), jnp.float32),
    )(x_p, W0, b0.reshape(1, H))


def _msg_call(hs, ea_aug, wfull, rmat, smat):
    return pl.pallas_call(
        _msg_body,
        grid=(E_PAD // TE,),
        in_specs=[pl.BlockSpec((TE, DN), lambda i: (i, 0)),
                  pl.BlockSpec((TE, DE + 1), lambda i: (i, 0)),
                  pl.BlockSpec((H, (DE + 1) * H), lambda i: (0, 0)),
                  pl.BlockSpec((DE + 1, (DE + 1) * H), lambda i: (0, 0)),
                  pl.BlockSpec(((DE + 1) * H, H), lambda i: (0, 0))],
        out_specs=pl.BlockSpec((TE, DN), lambda i: (i, 0)),
        out_shape=jax.ShapeDtypeStruct((E_PAD, DN), jnp.float32),
    )(hs, ea_aug, wfull, rmat, smat)


def _node_call(agg2, h, rw, cb, wih, whh, bih, bhh):
    return pl.pallas_call(
        _node_body,
        grid=(NP_ // TN,),
        in_specs=[pl.BlockSpec((TN, DN), lambda i: (i, 0)),
                  pl.BlockSpec((TN, DN), lambda i: (NP_ // TN + i, 0)),
                  pl.BlockSpec((TN, DN), lambda i: (i, 0)),
                  pl.BlockSpec((H, H), lambda i: (0, 0)),
                  pl.BlockSpec((1, H), lambda i: (0, 0)),
                  pl.BlockSpec((H, 3 * H), lambda i: (0, 0)),
                  pl.BlockSpec((H, 3 * H), lambda i: (0, 0)),
                  pl.BlockSpec((1, 3 * H), lambda i: (0, 0)),
                  pl.BlockSpec((1, 3 * H), lambda i: (0, 0))],
        out_specs=pl.BlockSpec((TN, DN), lambda i: (i, 0)),
        out_shape=jax.ShapeDtypeStruct((NP_, DN), jnp.float32),
    )(agg2, agg2, h, rw, cb, wih, whh, bih, bhh)


def _s2s_call(h, bt, wihT, whhT, bih, bhh, Wo, bo):
    return pl.pallas_call(
        _s2s_body,
        grid=(1,),
        in_specs=[pl.BlockSpec((NP_, DN), lambda i: (0, 0)),
                  pl.BlockSpec((NP_, 1), lambda i: (0, 0)),
                  pl.BlockSpec((2 * H, 4 * H), lambda i: (0, 0)),
                  pl.BlockSpec((H, 4 * H), lambda i: (0, 0)),
                  pl.BlockSpec((1, 4 * H), lambda i: (0, 0)),
                  pl.BlockSpec((1, 4 * H), lambda i: (0, 0)),
                  pl.BlockSpec((2 * H, H), lambda i: (0, 0)),
                  pl.BlockSpec((1, H), lambda i: (0, 0))],
        out_specs=pl.BlockSpec((B, H), lambda i: (0, 0)),
        out_shape=jax.ShapeDtypeStruct((B, H), jnp.float32),
    )(h, bt, wihT, whhT, bih, bhh, Wo, bo)


def kernel(x, edge_index, edge_attr, batch, W0, b0, edge_Wn, edge_bn,
           root_W, conv_b, gru_Wih, gru_Whh, gru_bih, gru_bhh,
           lstm_Wih, lstm_Whh, lstm_bih, lstm_bhh, Wo, bo):
    src = edge_index[0].astype(jnp.int32)
    dst = edge_index[1].astype(jnp.int32)
    epad = E_PAD - E
    src_p = jnp.concatenate([src, jnp.zeros((epad,), jnp.int32)])
    dst_p = jnp.concatenate([dst, jnp.zeros((epad,), jnp.int32)])
    ea_aug = jnp.concatenate(
        [edge_attr, jnp.ones((E, 1), jnp.float32)], axis=1)
    ea_aug = jnp.concatenate(
        [ea_aug, jnp.zeros((epad, DE + 1), jnp.float32)], axis=0)
    x_p = jnp.concatenate([x, jnp.zeros((NP_ - N, DN), jnp.float32)], axis=0)
    bt_p = jnp.concatenate(
        [batch.astype(jnp.int32), jnp.full((NP_ - N,), B, jnp.int32)]
    ).reshape(NP_, 1)
    wfull = []
    for l in range(L):
        w = edge_Wn[l].reshape(DE, H, H).transpose(1, 0, 2).reshape(H, DE * H)
        wfull.append(jnp.concatenate([w, edge_bn[l].reshape(H, H)], axis=1))
    zeros_n = jnp.zeros((NP_, DN), jnp.float32)
    dd = DE + 1
    rmat = jnp.asarray(
        _np.repeat(_np.eye(dd, dtype=_np.float32), H, axis=1))
    smat = jnp.asarray(
        _np.tile(_np.eye(H, dtype=_np.float32), (dd, 1)))

    src3 = src_p.reshape(NW, NCHUNK, CHUNK)
    dst3 = dst_p.reshape(NW, NCHUNK, CHUNK)
    sc_gather, sc_scatter = _sc_kernels()
    h = _h0_call(x_p, W0, b0)
    for l in range(L):
        hs = sc_gather(h, src3)
        msg = _msg_call(hs, ea_aug, wfull[l], rmat, smat)
        agg2 = sc_scatter(msg, dst3, zeros_n)
        h = _node_call(agg2, h, root_W[l], conv_b[l].reshape(1, H),
                       gru_Wih[l].T, gru_Whh[l].T,
                       gru_bih[l].reshape(1, 3 * H),
                       gru_bhh[l].reshape(1, 3 * H))
    return _s2s_call(h, bt_p, lstm_Wih.T, lstm_Whh.T,
                     lstm_bih.reshape(1, 4 * H), lstm_bhh.reshape(1, 4 * H),
                     Wo, bo.reshape(1, H))


# speedup vs baseline: 1.1048x; 1.1048x over previous
import functools

import numpy as _np

import jax
import jax.numpy as jnp
from jax import lax
from jax.experimental import pallas as pl
from jax.experimental.pallas import tpu as pltpu
from jax.experimental.pallas import tpu_sc as plsc

N = 10000
E = 160000
DN = 128
DE = 16
H = 32
L = 3
B = 64
STEPS = 6

NP_ = 10240
NC = 2
NS = 16
NW = NC * NS
E_PAD = 163840
EPW = E_PAD // NW
CHUNK = 128
NCHUNK = EPW // CHUNK
NBUF = 5
NH = 2
EH = E_PAD // NH
EPW_H = EH // NW
NCHUNK_H = EPW_H // CHUNK
NBUF_S = 2
TE = 2048
TN = 1024
ROWS_PER_TILE = NP_ // NS

@functools.lru_cache(maxsize=None)
def _sc_kernels(nchunk):
    epw = nchunk * CHUNK
    e_tot = NW * epw
    mesh = plsc.VectorSubcoreMesh(core_axis_name="c", subcore_axis_name="s")

    @functools.partial(
        pl.kernel, mesh=mesh,
        out_type=jax.ShapeDtypeStruct((NW * nchunk * CHUNK, DN), jnp.float32),
        scratch_types=[
            pltpu.VMEM((nchunk, CHUNK), jnp.int32),
            pltpu.VMEM((NBUF, CHUNK, DN), jnp.float32),
            pltpu.SemaphoreType.DMA,
            pltpu.SemaphoreType.DMA,
        ],
    )
    def sc_gather(h_hbm, src_hbm, out_hbm, idx_all, bufs, sem_g, sem_o):
        wid = lax.axis_index("s") * NC + lax.axis_index("c")
        base = wid * epw
        pltpu.sync_copy(src_hbm.at[wid], idx_all)

        def body(g, carry):
            j0 = g * NBUF
            cps = [
                pltpu.async_copy(
                    h_hbm.at[idx_all.at[j0 + b]],
                    bufs.at[b], sem_g)
                for b in range(NBUF)
            ]
            outs = []
            for b in range(NBUF):
                cps[b].wait()
                outs.append(pltpu.async_copy(
                    bufs.at[b],
                    out_hbm.at[pl.ds(base + (j0 + b) * CHUNK, CHUNK)],
                    sem_o))
            for o in outs:
                o.wait()
            return carry

        lax.fori_loop(0, nchunk // NBUF, body, 0)

    @functools.partial(
        pl.kernel, mesh=mesh,
        out_type=jax.ShapeDtypeStruct((NC * NP_, DN), jnp.float32),
        scratch_types=[
            pltpu.VMEM((nchunk, CHUNK), jnp.int32),
            pltpu.VMEM((NBUF_S, CHUNK, DN), jnp.float32),
            pltpu.VMEM_SHARED((NP_, DN), jnp.float32),
            pltpu.SemaphoreType.DMA,
            pltpu.SemaphoreType.DMA,
        ],
    )
    def sc_scatter(msg_hbm, dst_hbm, zeros_hbm, out_hbm, idx_all, bufs,
                   acc_sh, sem_l, sem_s):
        cid = lax.axis_index("c")
        sid = lax.axis_index("s")
        wid = sid * NC + cid
        r0 = sid * ROWS_PER_TILE
        pltpu.sync_copy(zeros_hbm.at[pl.ds(r0, ROWS_PER_TILE)],
                        acc_sh.at[pl.ds(r0, ROWS_PER_TILE)])
        base = wid * epw
        pltpu.sync_copy(dst_hbm.at[wid], idx_all)
        plsc.subcore_barrier()

        def body(g, carry):
            j0 = g * NBUF_S
            cps = [
                pltpu.async_copy(
                    msg_hbm.at[pl.ds(base + (j0 + b) * CHUNK, CHUNK)],
                    bufs.at[b], sem_l)
                for b in range(NBUF_S)
            ]
            adds = []
            for b in range(NBUF_S):
                cps[b].wait()
                adds.append(pltpu.async_copy(
                    bufs.at[b],
                    acc_sh.at[idx_all.at[j0 + b]],
                    sem_s, add=True))
            for a in adds:
                a.wait()
            return carry

        lax.fori_loop(0, nchunk // NBUF_S, body, 0)
        plsc.subcore_barrier()
        pltpu.sync_copy(acc_sh.at[pl.ds(r0, ROWS_PER_TILE)],
                        out_hbm.at[pl.ds(cid * NP_ + r0, ROWS_PER_TILE)])

    return sc_gather, sc_scatter


def _pad_wide(v):
    return jnp.concatenate(
        [v, jnp.zeros((v.shape[0], DN - H), jnp.float32)], axis=1)


def _h0_body(x_ref, w_ref, b_ref, o_ref):
    o_ref[...] = _pad_wide(jnp.dot(x_ref[...], w_ref[...],
                                   preferred_element_type=jnp.float32)
                           + b_ref[...])


def _msg_body(hs_ref, ea_ref, w_ref, r_ref, s_ref, o_ref):
    bf = jnp.bfloat16
    hs2 = jnp.dot(hs_ref[:, :H].astype(bf), w_ref[...].astype(bf),
                  preferred_element_type=jnp.float32)
    eax = jnp.dot(ea_ref[...].astype(bf), r_ref[...].astype(bf),
                  preferred_element_type=jnp.float32)
    msg = jnp.dot((eax * hs2).astype(bf), s_ref[...].astype(bf),
                  preferred_element_type=jnp.float32)
    o_ref[...] = _pad_wide(msg)


def _node_body(agg0_ref, agg1_ref, agg2_ref, agg3_ref, h_ref, rw_ref,
               cb_ref, wih_ref, whh_ref, bih_ref, bhh_ref, o_ref):
    h = h_ref[:, :H]
    agg = (agg0_ref[:, :H] + agg1_ref[:, :H]
           + agg2_ref[:, :H] + agg3_ref[:, :H])
    m = jnp.maximum(agg + jnp.dot(h, rw_ref[...],
                                  preferred_element_type=jnp.float32)
                    + cb_ref[...], 0.0)
    gi = jnp.dot(m, wih_ref[...], preferred_element_type=jnp.float32) + bih_ref[...]
    gh = jnp.dot(h, whh_ref[...], preferred_element_type=jnp.float32) + bhh_ref[...]
    r = jax.nn.sigmoid(gi[:, :H] + gh[:, :H])
    z = jax.nn.sigmoid(gi[:, H:2 * H] + gh[:, H:2 * H])
    n = jnp.tanh(gi[:, 2 * H:] + r * gh[:, 2 * H:])
    o_ref[...] = _pad_wide((1.0 - z) * n + z * h)


_NEG = -3.0e38


def _s2s_body(h_ref, bt_ref, wih_ref, whh_ref, bih_ref, bhh_ref,
              wo_ref, bo_ref, o_ref):
    h = h_ref[:, :H]
    bt = bt_ref[...]
    mm = (bt == lax.broadcasted_iota(jnp.int32, (NP_, B), 1)).astype(jnp.float32)
    q_star = jnp.zeros((B, 2 * H), jnp.float32)
    hl = jnp.zeros((B, H), jnp.float32)
    cl = jnp.zeros((B, H), jnp.float32)
    for _ in range(STEPS):
        gates = (jnp.dot(q_star, wih_ref[...], preferred_element_type=jnp.float32)
                 + bih_ref[...]
                 + jnp.dot(hl, whh_ref[...], preferred_element_type=jnp.float32)
                 + bhh_ref[...])
        gi = jax.nn.sigmoid(gates[:, :H])
        gf = jax.nn.sigmoid(gates[:, H:2 * H])
        gg = jnp.tanh(gates[:, 2 * H:3 * H])
        go = jax.nn.sigmoid(gates[:, 3 * H:])
        cl = gf * cl + gi * gg
        hl = go * jnp.tanh(cl)
        q = hl
        hq = lax.dot_general(h, q, (((1,), (1,)), ((), ())),
                             preferred_element_type=jnp.float32)
        e = jnp.sum(mm * hq, axis=1, keepdims=True)
        emax = jnp.max(jnp.where(mm > 0, e, _NEG), axis=0, keepdims=True)
        emax = jnp.where(emax > _NEG * 0.5, emax, 0.0)
        ex = jnp.exp(e - jnp.sum(mm * emax, axis=1, keepdims=True))
        denom = jnp.sum(mm * ex, axis=0, keepdims=True)
        a = ex / (jnp.sum(mm * denom, axis=1, keepdims=True) + 1e-16)
        r_ = lax.dot_general(mm * a, h, (((0,), (0,)), ((), ())),
                             preferred_element_type=jnp.float32)
        q_star = jnp.concatenate([q, r_], axis=-1)
    o_ref[...] = (jnp.dot(q_star, wo_ref[...],
                          preferred_element_type=jnp.float32) + bo_ref[...])


def _h0_call(x_p, W0, b0):
    return pl.pallas_call(
        _h0_body,
        grid=(NP_ // TN,),
        in_specs=[pl.BlockSpec((TN, DN), lambda i: (i, 0)),
                  pl.BlockSpec((DN, H), lambda i: (0, 0)),
                  pl.BlockSpec((1, H), lambda i: (0, 0))],
        out_specs=pl.BlockSpec((TN, DN), lambda i: (i, 0)),
        out_shape=jax.ShapeDtypeStruct((NP_, DN), jnp.float32),
    )(x_p, W0, b0.reshape(1, H))


def _msg_call(hs, ea_aug, wfull, rmat, smat):
    ne = hs.shape[0]
    return pl.pallas_call(
        _msg_body,
        grid=(ne // TE,),
        in_specs=[pl.BlockSpec((TE, DN), lambda i: (i, 0)),
                  pl.BlockSpec((TE, DE + 1), lambda i: (i, 0)),
                  pl.BlockSpec((H, (DE + 1) * H), lambda i: (0, 0)),
                  pl.BlockSpec((DE + 1, (DE + 1) * H), lambda i: (0, 0)),
                  pl.BlockSpec(((DE + 1) * H, H), lambda i: (0, 0))],
        out_specs=pl.BlockSpec((TE, DN), lambda i: (i, 0)),
        out_shape=jax.ShapeDtypeStruct((ne, DN), jnp.float32),
    )(hs, ea_aug, wfull, rmat, smat)


def _node_call(aggA, aggB, h, rw, cb, wih, whh, bih, bhh):
    return pl.pallas_call(
        _node_body,
        grid=(NP_ // TN,),
        in_specs=[pl.BlockSpec((TN, DN), lambda i: (i, 0)),
                  pl.BlockSpec((TN, DN), lambda i: (NP_ // TN + i, 0)),
                  pl.BlockSpec((TN, DN), lambda i: (i, 0)),
                  pl.BlockSpec((TN, DN), lambda i: (NP_ // TN + i, 0)),
                  pl.BlockSpec((TN, DN), lambda i: (i, 0)),
                  pl.BlockSpec((H, H), lambda i: (0, 0)),
                  pl.BlockSpec((1, H), lambda i: (0, 0)),
                  pl.BlockSpec((H, 3 * H), lambda i: (0, 0)),
                  pl.BlockSpec((H, 3 * H), lambda i: (0, 0)),
                  pl.BlockSpec((1, 3 * H), lambda i: (0, 0)),
                  pl.BlockSpec((1, 3 * H), lambda i: (0, 0))],
        out_specs=pl.BlockSpec((TN, DN), lambda i: (i, 0)),
        out_shape=jax.ShapeDtypeStruct((NP_, DN), jnp.float32),
    )(aggA, aggA, aggB, aggB, h, rw, cb, wih, whh, bih, bhh)


def _s2s_call(h, bt, wihT, whhT, bih, bhh, Wo, bo):
    return pl.pallas_call(
        _s2s_body,
        grid=(1,),
        in_specs=[pl.BlockSpec((NP_, DN), lambda i: (0, 0)),
                  pl.BlockSpec((NP_, 1), lambda i: (0, 0)),
                  pl.BlockSpec((2 * H, 4 * H), lambda i: (0, 0)),
                  pl.BlockSpec((H, 4 * H), lambda i: (0, 0)),
                  pl.BlockSpec((1, 4 * H), lambda i: (0, 0)),
                  pl.BlockSpec((1, 4 * H), lambda i: (0, 0)),
                  pl.BlockSpec((2 * H, H), lambda i: (0, 0)),
                  pl.BlockSpec((1, H), lambda i: (0, 0))],
        out_specs=pl.BlockSpec((B, H), lambda i: (0, 0)),
        out_shape=jax.ShapeDtypeStruct((B, H), jnp.float32),
    )(h, bt, wihT, whhT, bih, bhh, Wo, bo)


def kernel(x, edge_index, edge_attr, batch, W0, b0, edge_Wn, edge_bn,
           root_W, conv_b, gru_Wih, gru_Whh, gru_bih, gru_bhh,
           lstm_Wih, lstm_Whh, lstm_bih, lstm_bhh, Wo, bo):
    src = edge_index[0].astype(jnp.int32)
    dst = edge_index[1].astype(jnp.int32)
    epad = E_PAD - E
    src_p = jnp.concatenate([src, jnp.zeros((epad,), jnp.int32)])
    dst_p = jnp.concatenate([dst, jnp.zeros((epad,), jnp.int32)])
    ea_aug = jnp.concatenate(
        [edge_attr, jnp.ones((E, 1), jnp.float32)], axis=1)
    ea_aug = jnp.concatenate(
        [ea_aug, jnp.zeros((epad, DE + 1), jnp.float32)], axis=0)
    x_p = jnp.concatenate([x, jnp.zeros((NP_ - N, DN), jnp.float32)], axis=0)
    bt_p = jnp.concatenate(
        [batch.astype(jnp.int32), jnp.full((NP_ - N,), B, jnp.int32)]
    ).reshape(NP_, 1)
    wfull = []
    for l in range(L):
        w = edge_Wn[l].reshape(DE, H, H).transpose(1, 0, 2).reshape(H, DE * H)
        wfull.append(jnp.concatenate([w, edge_bn[l].reshape(H, H)], axis=1))
    zeros_n = jnp.zeros((NP_, DN), jnp.float32)
    dd = DE + 1
    rmat = jnp.asarray(
        _np.repeat(_np.eye(dd, dtype=_np.float32), H, axis=1))
    smat = jnp.asarray(
        _np.tile(_np.eye(H, dtype=_np.float32), (dd, 1)))

    srcH = [src_p[i * EH:(i + 1) * EH].reshape(NW, NCHUNK_H, CHUNK)
            for i in range(NH)]
    dstH = [dst_p[i * EH:(i + 1) * EH].reshape(NW, NCHUNK_H, CHUNK)
            for i in range(NH)]
    eaH = [ea_aug[i * EH:(i + 1) * EH] for i in range(NH)]
    sc_gather, sc_scatter = _sc_kernels(NCHUNK_H)
    h = _h0_call(x_p, W0, b0)
    for l in range(L):
        hsA = sc_gather(h, srcH[0])
        msgA = _msg_call(hsA, eaH[0], wfull[l], rmat, smat)
        hsB = sc_gather(h, srcH[1])
        msgB = _msg_call(hsB, eaH[1], wfull[l], rmat, smat)
        aggA = sc_scatter(msgA, dstH[0], zeros_n)
        aggB = sc_scatter(msgB, dstH[1], zeros_n)
        h = _node_call(aggA, aggB, h, root_W[l], conv_b[l].reshape(1, H),
                       gru_Wih[l].T, gru_Whh[l].T,
                       gru_bih[l].reshape(1, 3 * H),
                       gru_bhh[l].reshape(1, 3 * H))
    return _s2s_call(h, bt_p, lstm_Wih.T, lstm_Whh.T,
                     lstm_bih.reshape(1, 4 * H), lstm_bhh.reshape(1, 4 * H),
                     Wo, bo.reshape(1, H))
